# Initial kernel scaffold; baseline (speedup 1.0000x reference)
import jax
import jax.numpy as jnp
from jax.experimental import pallas as pl


def kernel(source_features, target_features, W_tsdm, b_tsdm, W_gddm, b_gddm):
    raise NotImplementedError("write your pallas kernel here")


# speedup vs baseline: 3.3515x; 3.3515x over previous
import functools

import jax
import jax.numpy as jnp
from jax.experimental import pallas as pl
from jax.experimental.pallas import tpu as pltpu

_N = 4096
_D = 32
_R = 256
_NB = _N // _R
_KP1 = 11
_BIG = 1e30


def _tgloss_body(src_ref, tgt_ref, wt_ref, bt_ref, wg_ref, bg_ref,
                 gdd_ref, disc_ref, sq_ref):
    i = pl.program_id(0)

    @pl.when(i == 0)
    def _init():
        x = tgt_ref[...]
        sq_ref[0, :] = jnp.sum(x * x, axis=1)
        mu_s = jnp.mean(src_ref[...], axis=0)
        mu_t = jnp.mean(x, axis=0)
        diff = mu_s - mu_t
        gdd_ref[0, 0] = jnp.abs(jnp.sum(diff * wg_ref[0, :]) + bg_ref[0])
        disc_ref[0, 0] = 0.0

    x = tgt_ref[...]
    xb = tgt_ref[pl.ds(i * _R, _R), :]
    sq = sq_ref[0, :]
    sqb = jnp.sum(xb * xb, axis=1)
    g = jax.lax.dot_general(xb, x, (((1,), (1,)), ((), ())),
                            preferred_element_type=jnp.float32)
    d2 = sqb[:, None] + sq[None, :] - 2.0 * g
    d2 = jnp.maximum(d2, 0.0)
    dist = jnp.where(d2 > 0.0, jnp.sqrt(jnp.where(d2 > 0.0, d2, 1.0)), 0.0)

    rowsum = jnp.sum(dist, axis=1, keepdims=True)

    t_prev = jnp.full((_R, 1), -1.0, jnp.float32)
    ts, cnts = [], []
    for _ in range(_KP1):
        masked = jnp.where(dist > t_prev, dist, _BIG)
        tj = jnp.min(masked, axis=1, keepdims=True)
        cj = jnp.sum(jnp.where(dist <= tj, 1.0, 0.0), axis=1, keepdims=True)
        ts.append(tj)
        cnts.append(cj)
        t_prev = tj
    tstar = jnp.full((_R, 1), _BIG, jnp.float32)
    for j in range(_KP1):
        tstar = jnp.minimum(tstar, jnp.where(cnts[j] >= _KP1, ts[j], _BIG))
    lt = dist < tstar
    s_lt = jnp.sum(jnp.where(lt, dist, 0.0), axis=1, keepdims=True)
    c_lt = jnp.sum(jnp.where(lt, 1.0, 0.0), axis=1, keepdims=True)
    s11 = s_lt + tstar * (_KP1 - c_lt)
    inc_comp = (s11 - ts[0]) / 10.0
    inc_sep = rowsum / (_N - 1)
    disc = jnp.abs(inc_comp * wt_ref[0, 0] + inc_sep * wt_ref[0, 1]
                   + bt_ref[0])
    disc_ref[0, 0] += jnp.sum(disc)


@functools.partial(jax.jit, static_argnames=())
def kernel(source_features, target_features, W_tsdm, b_tsdm, W_gddm, b_gddm):
    gdd, disc = pl.pallas_call(
        _tgloss_body,
        grid=(_NB,),
        in_specs=[
            pl.BlockSpec((_N, _D), lambda i: (0, 0)),
            pl.BlockSpec((_N, _D), lambda i: (0, 0)),
            pl.BlockSpec(memory_space=pltpu.SMEM),
            pl.BlockSpec(memory_space=pltpu.SMEM),
            pl.BlockSpec((1, _D), lambda i: (0, 0)),
            pl.BlockSpec(memory_space=pltpu.SMEM),
        ],
        out_specs=[
            pl.BlockSpec(memory_space=pltpu.SMEM),
            pl.BlockSpec(memory_space=pltpu.SMEM),
        ],
        out_shape=[
            jax.ShapeDtypeStruct((1, 1), jnp.float32),
            jax.ShapeDtypeStruct((1, 1), jnp.float32),
        ],
        scratch_shapes=[pltpu.VMEM((1, _N), jnp.float32)],
        compiler_params=pltpu.CompilerParams(
            dimension_semantics=("arbitrary",),
        ),
    )(source_features, target_features, W_tsdm, b_tsdm, W_gddm, b_gddm)
    return (gdd[0, 0], disc[0, 0] / _N)
